# Initial kernel scaffold; baseline (speedup 1.0000x reference)
import jax
import jax.numpy as jnp
from jax.experimental import pallas as pl


def kernel(x, edge_index, batch, params):
    raise NotImplementedError("write your pallas kernel here")


# speedup vs baseline: 6.3007x; 6.3007x over previous
import functools

import jax
import jax.numpy as jnp
from jax import lax
from jax.experimental import pallas as pl
from jax.experimental.pallas import tpu as pltpu
from jax.experimental.pallas import tpu_sc as plsc

N_NODES = 10000
N_EDGES = 320000
IN_DIM = 128
HID = 64
OUT = 3
NLAYERS = 3
NGRAPHS = 128
BN_EPS = 1e-5

_NC = 2
_NS = 16
_NW = _NC * _NS
_CH = 128
_NCHUNKS = N_EDGES // _CH
_CHUNKS_PER_W = _NCHUNKS // _NW
_CHUNK_REM = _NCHUNKS - _CHUNKS_PER_W * _NW
_RPT = 624
_RTAIL = N_NODES - _RPT * _NS


def _segsum_sc(h, ei, zeros):
    mesh = plsc.VectorSubcoreMesh(core_axis_name="c", subcore_axis_name="s")

    @functools.partial(
        pl.kernel,
        mesh=mesh,
        compiler_params=pltpu.CompilerParams(use_tc_tiling_on_sc=False),
        out_type=jax.ShapeDtypeStruct((_NC, N_NODES, HID), jnp.float32),
        scratch_types=[
            pltpu.VMEM((_CH,), jnp.int32),
            pltpu.VMEM((_CH,), jnp.int32),
            pltpu.VMEM((_CH, HID), jnp.float32),
            pltpu.VMEM_SHARED((N_NODES, HID), jnp.float32),
            pltpu.SemaphoreType.DMA,
        ],
    )
    def k(h_hbm, ei_hbm, z_hbm, out_hbm, src_v, dst_v, rows_v, acc_sh, sem):
        c = lax.axis_index("c")
        s = lax.axis_index("s")
        wid = s * _NC + c
        pltpu.sync_copy(z_hbm.at[pl.ds(s * _RPT, _RPT)],
                        acc_sh.at[pl.ds(s * _RPT, _RPT)])

        @pl.when(s == 0)
        def _():
            pltpu.sync_copy(z_hbm.at[pl.ds(_RPT * _NS, _RTAIL)],
                            acc_sh.at[pl.ds(_RPT * _NS, _RTAIL)])

        plsc.subcore_barrier()

        first = wid * _CHUNKS_PER_W + jnp.minimum(wid, _CHUNK_REM)
        n = _CHUNKS_PER_W + jnp.where(wid < _CHUNK_REM, 1, 0)

        def body(i, carry):
            off = (first + i) * _CH
            pltpu.sync_copy(ei_hbm.at[0, pl.ds(off, _CH)], src_v)
            pltpu.sync_copy(ei_hbm.at[1, pl.ds(off, _CH)], dst_v)
            pltpu.async_copy(h_hbm.at[src_v], rows_v, sem).wait()
            pltpu.sync_copy(rows_v, acc_sh.at[dst_v], add=True)
            return carry

        lax.fori_loop(0, n, body, 0)
        plsc.subcore_barrier()
        pltpu.sync_copy(acc_sh.at[pl.ds(s * _RPT, _RPT)],
                        out_hbm.at[c, pl.ds(s * _RPT, _RPT)])

        @pl.when(s == 0)
        def _():
            pltpu.sync_copy(acc_sh.at[pl.ds(_RPT * _NS, _RTAIL)],
                            out_hbm.at[c, pl.ds(_RPT * _NS, _RTAIL)])

    return k(h, ei, zeros)


def _tc_feat(x, w, b):
    def body(x_ref, w_ref, b_ref, o_ref):
        o_ref[...] = jnp.maximum(
            jnp.dot(x_ref[...], w_ref[...], preferred_element_type=jnp.float32)
            + b_ref[...], 0.0)

    return pl.pallas_call(
        body,
        out_shape=jax.ShapeDtypeStruct((N_NODES, HID), jnp.float32),
    )(x, w, b)


def _tc_mlp(h, agg, w1, b1, w2, b2, s3, b3, epsp):
    def body(h_ref, a_ref, w1_ref, b1_ref, w2_ref, b2_ref, s3_ref, b3_ref,
             e_ref, o_ref):
        z = h_ref[...] * e_ref[0, 0] + a_ref[0] + a_ref[1]
        z = jnp.maximum(
            jnp.dot(z, w1_ref[...], preferred_element_type=jnp.float32)
            + b1_ref[...], 0.0)
        z = jnp.maximum(
            jnp.dot(z, w2_ref[...], preferred_element_type=jnp.float32)
            + b2_ref[...], 0.0)
        o_ref[...] = jnp.maximum(z * s3_ref[...] + b3_ref[...], 0.0)

    return pl.pallas_call(
        body,
        out_shape=jax.ShapeDtypeStruct((N_NODES, HID), jnp.float32),
    )(h, agg, w1, b1, w2, b2, s3, b3, epsp)


def _tc_head(h, batch2d, a1w, a1b, a2w, a2b, a3w, a3b, tempp,
             c1w, c1b, c2w, c2b):
    def body(h_ref, b_ref, a1w_ref, a1b_ref, a2w_ref, a2b_ref, a3w_ref,
             a3b_ref, t_ref, c1w_ref, c1b_ref, c2w_ref, c2b_ref,
             lo_ref, sc_ref):
        h_ = h_ref[...]
        a = jnp.dot(h_, a1w_ref[...], preferred_element_type=jnp.float32) \
            + a1b_ref[...]
        a = jnp.where(a > 0, a, 0.2 * a)
        a = jnp.dot(a, a2w_ref[...], preferred_element_type=jnp.float32) \
            + a2b_ref[...]
        a = jnp.where(a > 0, a, 0.2 * a)
        lg = jnp.sum(a * a3w_ref[...], axis=1, keepdims=True) + a3b_ref[...]
        lg = jnp.clip(lg, -10.0, 10.0)
        s = jax.nn.sigmoid(lg * t_ref[0, 0])
        sc_ref[...] = s
        onehot = (b_ref[...] == lax.broadcasted_iota(
            jnp.int32, (N_NODES, NGRAPHS), 1)).astype(jnp.float32)
        ge = lax.dot_general(onehot, h_ * s, (((0,), (0,)), ((), ())),
                             preferred_element_type=jnp.float32)
        c1 = jnp.maximum(
            jnp.dot(ge, c1w_ref[...], preferred_element_type=jnp.float32)
            + c1b_ref[...], 0.0)
        lo_ref[...] = jnp.dot(c1, c2w_ref[...],
                              preferred_element_type=jnp.float32) + c2b_ref[...]

    return pl.pallas_call(
        body,
        out_shape=(
            jax.ShapeDtypeStruct((NGRAPHS, OUT), jnp.float32),
            jax.ShapeDtypeStruct((N_NODES, 1), jnp.float32),
        ),
    )(h, batch2d, a1w, a1b, a2w, a2b, a3w, a3b, tempp, c1w, c1b, c2w, c2b)


def kernel(x, edge_index, batch, params):
    ei = edge_index.astype(jnp.int32)
    bn_s = 1.0 / jnp.sqrt(1.0 + BN_EPS)

    h = _tc_feat(x, params["feat"]["w"], params["feat"]["b"].reshape(1, HID))

    zeros = jnp.zeros((N_NODES, HID), jnp.float32)
    for c in params["convs"]:
        g1 = c["bn1g"] * bn_s
        w1 = c["lin1"]["w"] * g1[None, :]
        b1 = (c["lin1"]["b"] * g1 + c["bn1b"]).reshape(1, HID)
        g2 = c["bn2g"] * bn_s
        w2 = c["lin2"]["w"] * g2[None, :]
        b2 = (c["lin2"]["b"] * g2 + c["bn2b"]).reshape(1, HID)
        s3 = (c["bng"] * bn_s).reshape(1, HID)
        b3 = c["bnb"].reshape(1, HID)
        epsp = (1.0 + c["eps"]).reshape(1, 1)

        agg = _segsum_sc(h, ei, zeros)
        h = _tc_mlp(h, agg, w1, b1, w2, b2, s3, b3, epsp)

    logits, scores = _tc_head(
        h,
        batch.reshape(N_NODES, 1),
        params["attn1"]["w"], params["attn1"]["b"].reshape(1, HID // 2),
        params["attn2"]["w"], params["attn2"]["b"].reshape(1, HID // 4),
        params["attn3"]["w"].reshape(1, HID // 4),
        params["attn3"]["b"].reshape(1, 1),
        params["temp"].reshape(1, 1),
        params["cls1"]["w"], params["cls1"]["b"].reshape(1, HID // 2),
        params["cls2"]["w"], params["cls2"]["b"].reshape(1, OUT),
    )
    return logits, scores


# speedup vs baseline: 13.7654x; 2.1847x over previous
import functools

import jax
import jax.numpy as jnp
from jax import lax
from jax.experimental import pallas as pl
from jax.experimental.pallas import tpu as pltpu
from jax.experimental.pallas import tpu_sc as plsc

N_NODES = 10000
N_EDGES = 320000
IN_DIM = 128
HID = 64
OUT = 3
NLAYERS = 3
NGRAPHS = 128
BN_EPS = 1e-5

_NC = 2
_NS = 16
_NW = _NC * _NS
_CH = 128
_NCHUNKS = N_EDGES // _CH
_CHUNKS_PER_W = _NCHUNKS // _NW
_CHUNK_REM = _NCHUNKS - _CHUNKS_PER_W * _NW
_RPT = 624
_RTAIL = N_NODES - _RPT * _NS


_NBUF = 4


def _segsum_sc(h, ei3, zeros):
    mesh = plsc.VectorSubcoreMesh(core_axis_name="c", subcore_axis_name="s")

    @functools.partial(
        pl.kernel,
        mesh=mesh,
        compiler_params=pltpu.CompilerParams(use_tc_tiling_on_sc=False),
        out_type=jax.ShapeDtypeStruct((_NC, N_NODES, HID), jnp.float32),
        scratch_types=[
            pltpu.VMEM((_CHUNKS_PER_W + 1, _CH), jnp.int32),
            pltpu.VMEM((_CHUNKS_PER_W + 1, _CH), jnp.int32),
            pltpu.VMEM((_NBUF, _CH, HID), jnp.float32),
            pltpu.VMEM_SHARED((N_NODES, HID), jnp.float32),
            pltpu.SemaphoreType.DMA((_NBUF,)),
            pltpu.SemaphoreType.DMA((_NBUF,)),
        ],
    )
    def k(h_hbm, ei_hbm, z_hbm, out_hbm, src_v, dst_v, rows_v, acc_sh,
          gsem, ssem):
        c = lax.axis_index("c")
        s = lax.axis_index("s")
        wid = s * _NC + c
        pltpu.sync_copy(z_hbm.at[pl.ds(s * _RPT, _RPT)],
                        acc_sh.at[pl.ds(s * _RPT, _RPT)])

        @pl.when(s == 0)
        def _():
            pltpu.sync_copy(z_hbm.at[pl.ds(_RPT * _NS, _RTAIL)],
                            acc_sh.at[pl.ds(_RPT * _NS, _RTAIL)])

        first = wid * _CHUNKS_PER_W + jnp.minimum(wid, _CHUNK_REM)
        n = _CHUNKS_PER_W + jnp.where(wid < _CHUNK_REM, 1, 0)

        pltpu.sync_copy(ei_hbm.at[0, pl.ds(first, _CHUNKS_PER_W)],
                        src_v.at[pl.ds(0, _CHUNKS_PER_W)])
        pltpu.sync_copy(ei_hbm.at[1, pl.ds(first, _CHUNKS_PER_W)],
                        dst_v.at[pl.ds(0, _CHUNKS_PER_W)])

        @pl.when(wid < _CHUNK_REM)
        def _():
            pltpu.sync_copy(ei_hbm.at[0, pl.ds(first + _CHUNKS_PER_W, 1)],
                            src_v.at[pl.ds(_CHUNKS_PER_W, 1)])
            pltpu.sync_copy(ei_hbm.at[1, pl.ds(first + _CHUNKS_PER_W, 1)],
                            dst_v.at[pl.ds(_CHUNKS_PER_W, 1)])

        plsc.subcore_barrier()

        pltpu.async_copy(h_hbm.at[src_v.at[0]], rows_v.at[0], gsem.at[0])
        pltpu.async_copy(h_hbm.at[src_v.at[1]], rows_v.at[1], gsem.at[1])

        def body(i, carry):
            b = lax.rem(i, _NBUF)
            pltpu.make_async_copy(h_hbm.at[src_v.at[i]], rows_v.at[b],
                                  gsem.at[b]).wait()
            pltpu.async_copy(rows_v.at[b], acc_sh.at[dst_v.at[i]],
                             ssem.at[b], add=True)

            @pl.when(i + 2 < n)
            def _():
                bj = lax.rem(i + 2, _NBUF)

                @pl.when(i >= 2)
                def _():
                    pltpu.make_async_copy(rows_v.at[bj],
                                          acc_sh.at[dst_v.at[0]],
                                          ssem.at[bj]).wait()

                pltpu.async_copy(h_hbm.at[src_v.at[i + 2]], rows_v.at[bj],
                                 gsem.at[bj])

            return carry

        lax.fori_loop(0, n, body, 0)
        for d in range(1, _NBUF + 1):
            pltpu.make_async_copy(rows_v.at[lax.rem(n - d, _NBUF)],
                                  acc_sh.at[dst_v.at[0]],
                                  ssem.at[lax.rem(n - d, _NBUF)]).wait()
        plsc.subcore_barrier()
        pltpu.sync_copy(acc_sh.at[pl.ds(s * _RPT, _RPT)],
                        out_hbm.at[c, pl.ds(s * _RPT, _RPT)])

        @pl.when(s == 0)
        def _():
            pltpu.sync_copy(acc_sh.at[pl.ds(_RPT * _NS, _RTAIL)],
                            out_hbm.at[c, pl.ds(_RPT * _NS, _RTAIL)])

    return k(h, ei3, zeros)


def _tc_feat(x, w, b):
    def body(x_ref, w_ref, b_ref, o_ref):
        o_ref[...] = jnp.maximum(
            jnp.dot(x_ref[...], w_ref[...], preferred_element_type=jnp.float32)
            + b_ref[...], 0.0)

    return pl.pallas_call(
        body,
        out_shape=jax.ShapeDtypeStruct((N_NODES, HID), jnp.float32),
    )(x, w, b)


def _tc_mlp(h, agg, w1, b1, w2, b2, s3, b3, epsp):
    def body(h_ref, a_ref, w1_ref, b1_ref, w2_ref, b2_ref, s3_ref, b3_ref,
             e_ref, o_ref):
        z = h_ref[...] * e_ref[0, 0] + a_ref[0] + a_ref[1]
        z = jnp.maximum(
            jnp.dot(z, w1_ref[...], preferred_element_type=jnp.float32)
            + b1_ref[...], 0.0)
        z = jnp.maximum(
            jnp.dot(z, w2_ref[...], preferred_element_type=jnp.float32)
            + b2_ref[...], 0.0)
        o_ref[...] = jnp.maximum(z * s3_ref[...] + b3_ref[...], 0.0)

    return pl.pallas_call(
        body,
        out_shape=jax.ShapeDtypeStruct((N_NODES, HID), jnp.float32),
    )(h, agg, w1, b1, w2, b2, s3, b3, epsp)


def _tc_head(h, batch2d, a1w, a1b, a2w, a2b, a3w, a3b, tempp,
             c1w, c1b, c2w, c2b):
    def body(h_ref, b_ref, a1w_ref, a1b_ref, a2w_ref, a2b_ref, a3w_ref,
             a3b_ref, t_ref, c1w_ref, c1b_ref, c2w_ref, c2b_ref,
             lo_ref, sc_ref):
        h_ = h_ref[...]
        a = jnp.dot(h_, a1w_ref[...], preferred_element_type=jnp.float32) \
            + a1b_ref[...]
        a = jnp.where(a > 0, a, 0.2 * a)
        a = jnp.dot(a, a2w_ref[...], preferred_element_type=jnp.float32) \
            + a2b_ref[...]
        a = jnp.where(a > 0, a, 0.2 * a)
        lg = jnp.sum(a * a3w_ref[...], axis=1, keepdims=True) + a3b_ref[...]
        lg = jnp.clip(lg, -10.0, 10.0)
        s = jax.nn.sigmoid(lg * t_ref[0, 0])
        sc_ref[...] = s
        onehot = (b_ref[...] == lax.broadcasted_iota(
            jnp.int32, (N_NODES, NGRAPHS), 1)).astype(jnp.float32)
        ge = lax.dot_general(onehot, h_ * s, (((0,), (0,)), ((), ())),
                             preferred_element_type=jnp.float32)
        c1 = jnp.maximum(
            jnp.dot(ge, c1w_ref[...], preferred_element_type=jnp.float32)
            + c1b_ref[...], 0.0)
        lo_ref[...] = jnp.dot(c1, c2w_ref[...],
                              preferred_element_type=jnp.float32) + c2b_ref[...]

    return pl.pallas_call(
        body,
        out_shape=(
            jax.ShapeDtypeStruct((NGRAPHS, OUT), jnp.float32),
            jax.ShapeDtypeStruct((N_NODES, 1), jnp.float32),
        ),
    )(h, batch2d, a1w, a1b, a2w, a2b, a3w, a3b, tempp, c1w, c1b, c2w, c2b)


def kernel(x, edge_index, batch, params):
    ei = edge_index.astype(jnp.int32).reshape(2, _NCHUNKS, _CH)
    bn_s = 1.0 / jnp.sqrt(1.0 + BN_EPS)

    h = _tc_feat(x, params["feat"]["w"], params["feat"]["b"].reshape(1, HID))

    zeros = jnp.zeros((N_NODES, HID), jnp.float32)
    for c in params["convs"]:
        g1 = c["bn1g"] * bn_s
        w1 = c["lin1"]["w"] * g1[None, :]
        b1 = (c["lin1"]["b"] * g1 + c["bn1b"]).reshape(1, HID)
        g2 = c["bn2g"] * bn_s
        w2 = c["lin2"]["w"] * g2[None, :]
        b2 = (c["lin2"]["b"] * g2 + c["bn2b"]).reshape(1, HID)
        s3 = (c["bng"] * bn_s).reshape(1, HID)
        b3 = c["bnb"].reshape(1, HID)
        epsp = (1.0 + c["eps"]).reshape(1, 1)

        agg = _segsum_sc(h, ei, zeros)
        h = _tc_mlp(h, agg, w1, b1, w2, b2, s3, b3, epsp)

    logits, scores = _tc_head(
        h,
        batch.reshape(N_NODES, 1),
        params["attn1"]["w"], params["attn1"]["b"].reshape(1, HID // 2),
        params["attn2"]["w"], params["attn2"]["b"].reshape(1, HID // 4),
        params["attn3"]["w"].reshape(1, HID // 4),
        params["attn3"]["b"].reshape(1, 1),
        params["temp"].reshape(1, 1),
        params["cls1"]["w"], params["cls1"]["b"].reshape(1, HID // 2),
        params["cls2"]["w"], params["cls2"]["b"].reshape(1, OUT),
    )
    return logits, scores


# speedup vs baseline: 14.4862x; 1.0524x over previous
import functools

import jax
import jax.numpy as jnp
from jax import lax
from jax.experimental import pallas as pl
from jax.experimental.pallas import tpu as pltpu
from jax.experimental.pallas import tpu_sc as plsc

N_NODES = 10000
N_EDGES = 320000
IN_DIM = 128
HID = 64
OUT = 3
NLAYERS = 3
NGRAPHS = 128
BN_EPS = 1e-5

_NC = 2
_NS = 16
_NW = _NC * _NS
_CH = 128
_NCHUNKS = N_EDGES // _CH
_CHUNKS_PER_W = _NCHUNKS // _NW
_CHUNK_REM = _NCHUNKS - _CHUNKS_PER_W * _NW
_RPT = 624
_RTAIL = N_NODES - _RPT * _NS


_NBUF = 6


def _segsum_sc(h, ei3, zeros):
    mesh = plsc.VectorSubcoreMesh(core_axis_name="c", subcore_axis_name="s")

    @functools.partial(
        pl.kernel,
        mesh=mesh,
        compiler_params=pltpu.CompilerParams(use_tc_tiling_on_sc=False),
        out_type=jax.ShapeDtypeStruct((_NC, N_NODES, HID), jnp.float32),
        scratch_types=[
            pltpu.VMEM((_CHUNKS_PER_W + 1, _CH), jnp.int32),
            pltpu.VMEM((_CHUNKS_PER_W + 1, _CH), jnp.int32),
            pltpu.VMEM((_NBUF, _CH, HID), jnp.float32),
            pltpu.VMEM_SHARED((N_NODES, HID), jnp.float32),
            pltpu.SemaphoreType.DMA((_NBUF,)),
            pltpu.SemaphoreType.DMA((_NBUF,)),
        ],
    )
    def k(h_hbm, ei_hbm, z_hbm, out_hbm, src_v, dst_v, rows_v, acc_sh,
          gsem, ssem):
        c = lax.axis_index("c")
        s = lax.axis_index("s")
        wid = s * _NC + c
        pltpu.sync_copy(z_hbm.at[pl.ds(s * _RPT, _RPT)],
                        acc_sh.at[pl.ds(s * _RPT, _RPT)])

        @pl.when(s == 0)
        def _():
            pltpu.sync_copy(z_hbm.at[pl.ds(_RPT * _NS, _RTAIL)],
                            acc_sh.at[pl.ds(_RPT * _NS, _RTAIL)])

        first = wid * _CHUNKS_PER_W + jnp.minimum(wid, _CHUNK_REM)

        pltpu.sync_copy(ei_hbm.at[0, pl.ds(first, _CHUNKS_PER_W)],
                        src_v.at[pl.ds(0, _CHUNKS_PER_W)])
        pltpu.sync_copy(ei_hbm.at[1, pl.ds(first, _CHUNKS_PER_W)],
                        dst_v.at[pl.ds(0, _CHUNKS_PER_W)])

        @pl.when(wid < _CHUNK_REM)
        def _():
            pltpu.sync_copy(ei_hbm.at[0, pl.ds(first + _CHUNKS_PER_W, 1)],
                            src_v.at[pl.ds(_CHUNKS_PER_W, 1)])
            pltpu.sync_copy(ei_hbm.at[1, pl.ds(first + _CHUNKS_PER_W, 1)],
                            dst_v.at[pl.ds(_CHUNKS_PER_W, 1)])

        plsc.subcore_barrier()

        for b in range(_NBUF):
            pltpu.async_copy(h_hbm.at[src_v.at[b]], rows_v.at[b],
                             gsem.at[b])

        n_outer = _CHUNKS_PER_W // _NBUF

        def body(j, carry):
            i0 = j * _NBUF
            for b in range(_NBUF):
                pltpu.make_async_copy(h_hbm.at[src_v.at[i0 + b]],
                                      rows_v.at[b], gsem.at[b]).wait()
                pltpu.async_copy(rows_v.at[b], acc_sh.at[dst_v.at[i0 + b]],
                                 ssem.at[b], add=True)

            @pl.when(j + 1 < n_outer)
            def _():
                for b in range(_NBUF):
                    pltpu.make_async_copy(rows_v.at[b],
                                          acc_sh.at[dst_v.at[0]],
                                          ssem.at[b]).wait()
                    pltpu.async_copy(h_hbm.at[src_v.at[i0 + _NBUF + b]],
                                     rows_v.at[b], gsem.at[b])

            return carry

        lax.fori_loop(0, n_outer, body, 0)
        for b in range(_NBUF):
            pltpu.make_async_copy(rows_v.at[b], acc_sh.at[dst_v.at[0]],
                                  ssem.at[b]).wait()

        @pl.when(wid < _CHUNK_REM)
        def _():
            i = _CHUNKS_PER_W
            pltpu.async_copy(h_hbm.at[src_v.at[i]], rows_v.at[0],
                             gsem.at[0]).wait()
            pltpu.sync_copy(rows_v.at[0], acc_sh.at[dst_v.at[i]], add=True)

        plsc.subcore_barrier()
        pltpu.sync_copy(acc_sh.at[pl.ds(s * _RPT, _RPT)],
                        out_hbm.at[c, pl.ds(s * _RPT, _RPT)])

        @pl.when(s == 0)
        def _():
            pltpu.sync_copy(acc_sh.at[pl.ds(_RPT * _NS, _RTAIL)],
                            out_hbm.at[c, pl.ds(_RPT * _NS, _RTAIL)])

    return k(h, ei3, zeros)


def _tc_feat(x, w, b):
    def body(x_ref, w_ref, b_ref, o_ref):
        o_ref[...] = jnp.maximum(
            jnp.dot(x_ref[...], w_ref[...], preferred_element_type=jnp.float32)
            + b_ref[...], 0.0)

    return pl.pallas_call(
        body,
        out_shape=jax.ShapeDtypeStruct((N_NODES, HID), jnp.float32),
    )(x, w, b)


def _tc_mlp(h, agg, w1, b1, w2, b2, s3, b3, epsp):
    def body(h_ref, a_ref, w1_ref, b1_ref, w2_ref, b2_ref, s3_ref, b3_ref,
             e_ref, o_ref):
        z = h_ref[...] * e_ref[0, 0] + a_ref[0] + a_ref[1]
        z = jnp.maximum(
            jnp.dot(z, w1_ref[...], preferred_element_type=jnp.float32)
            + b1_ref[...], 0.0)
        z = jnp.maximum(
            jnp.dot(z, w2_ref[...], preferred_element_type=jnp.float32)
            + b2_ref[...], 0.0)
        o_ref[...] = jnp.maximum(z * s3_ref[...] + b3_ref[...], 0.0)

    return pl.pallas_call(
        body,
        out_shape=jax.ShapeDtypeStruct((N_NODES, HID), jnp.float32),
    )(h, agg, w1, b1, w2, b2, s3, b3, epsp)


def _tc_head(h, batch2d, a1w, a1b, a2w, a2b, a3w, a3b, tempp,
             c1w, c1b, c2w, c2b):
    def body(h_ref, b_ref, a1w_ref, a1b_ref, a2w_ref, a2b_ref, a3w_ref,
             a3b_ref, t_ref, c1w_ref, c1b_ref, c2w_ref, c2b_ref,
             lo_ref, sc_ref):
        h_ = h_ref[...]
        a = jnp.dot(h_, a1w_ref[...], preferred_element_type=jnp.float32) \
            + a1b_ref[...]
        a = jnp.where(a > 0, a, 0.2 * a)
        a = jnp.dot(a, a2w_ref[...], preferred_element_type=jnp.float32) \
            + a2b_ref[...]
        a = jnp.where(a > 0, a, 0.2 * a)
        lg = jnp.sum(a * a3w_ref[...], axis=1, keepdims=True) + a3b_ref[...]
        lg = jnp.clip(lg, -10.0, 10.0)
        s = jax.nn.sigmoid(lg * t_ref[0, 0])
        sc_ref[...] = s
        onehot = (b_ref[...] == lax.broadcasted_iota(
            jnp.int32, (N_NODES, NGRAPHS), 1)).astype(jnp.float32)
        ge = lax.dot_general(onehot, h_ * s, (((0,), (0,)), ((), ())),
                             preferred_element_type=jnp.float32)
        c1 = jnp.maximum(
            jnp.dot(ge, c1w_ref[...], preferred_element_type=jnp.float32)
            + c1b_ref[...], 0.0)
        lo_ref[...] = jnp.dot(c1, c2w_ref[...],
                              preferred_element_type=jnp.float32) + c2b_ref[...]

    return pl.pallas_call(
        body,
        out_shape=(
            jax.ShapeDtypeStruct((NGRAPHS, OUT), jnp.float32),
            jax.ShapeDtypeStruct((N_NODES, 1), jnp.float32),
        ),
    )(h, batch2d, a1w, a1b, a2w, a2b, a3w, a3b, tempp, c1w, c1b, c2w, c2b)


def kernel(x, edge_index, batch, params):
    ei = edge_index.astype(jnp.int32).reshape(2, _NCHUNKS, _CH)
    bn_s = 1.0 / jnp.sqrt(1.0 + BN_EPS)

    h = _tc_feat(x, params["feat"]["w"], params["feat"]["b"].reshape(1, HID))

    zeros = jnp.zeros((N_NODES, HID), jnp.float32)
    for c in params["convs"]:
        g1 = c["bn1g"] * bn_s
        w1 = c["lin1"]["w"] * g1[None, :]
        b1 = (c["lin1"]["b"] * g1 + c["bn1b"]).reshape(1, HID)
        g2 = c["bn2g"] * bn_s
        w2 = c["lin2"]["w"] * g2[None, :]
        b2 = (c["lin2"]["b"] * g2 + c["bn2b"]).reshape(1, HID)
        s3 = (c["bng"] * bn_s).reshape(1, HID)
        b3 = c["bnb"].reshape(1, HID)
        epsp = (1.0 + c["eps"]).reshape(1, 1)

        agg = _segsum_sc(h, ei, zeros)
        h = _tc_mlp(h, agg, w1, b1, w2, b2, s3, b3, epsp)

    logits, scores = _tc_head(
        h,
        batch.reshape(N_NODES, 1),
        params["attn1"]["w"], params["attn1"]["b"].reshape(1, HID // 2),
        params["attn2"]["w"], params["attn2"]["b"].reshape(1, HID // 4),
        params["attn3"]["w"].reshape(1, HID // 4),
        params["attn3"]["b"].reshape(1, 1),
        params["temp"].reshape(1, 1),
        params["cls1"]["w"], params["cls1"]["b"].reshape(1, HID // 2),
        params["cls2"]["w"], params["cls2"]["b"].reshape(1, OUT),
    )
    return logits, scores


# speedup vs baseline: 17.4701x; 1.2060x over previous
import functools

import jax
import jax.numpy as jnp
from jax import lax
from jax.experimental import pallas as pl
from jax.experimental.pallas import tpu as pltpu
from jax.experimental.pallas import tpu_sc as plsc

N_NODES = 10000
N_EDGES = 320000
IN_DIM = 128
HID = 64
OUT = 3
NLAYERS = 3
NGRAPHS = 128
BN_EPS = 1e-5

_NC = 2
_NS = 16
_NW = _NC * _NS
_CH = 128
_NCHUNKS = N_EDGES // _CH
_CHUNKS_PER_W = _NCHUNKS // _NW
_CHUNK_REM = _NCHUNKS - _CHUNKS_PER_W * _NW
_RPT = 624
_RTAIL = N_NODES - _RPT * _NS


_NBUF = 6


def _segsum_sc(h, ei3, zeros):
    mesh = plsc.VectorSubcoreMesh(core_axis_name="c", subcore_axis_name="s")

    @functools.partial(
        pl.kernel,
        mesh=mesh,
        compiler_params=pltpu.CompilerParams(use_tc_tiling_on_sc=False),
        out_type=jax.ShapeDtypeStruct((_NC, N_NODES, HID), jnp.float32),
        scratch_types=[
            pltpu.VMEM((_CHUNKS_PER_W + 1, _CH), jnp.int32),
            pltpu.VMEM((_CHUNKS_PER_W + 1, _CH), jnp.int32),
            pltpu.VMEM((_NBUF, _CH, HID), jnp.float32),
            pltpu.VMEM_SHARED((N_NODES, HID), jnp.float32),
            pltpu.SemaphoreType.DMA((_NBUF,)),
            pltpu.SemaphoreType.DMA((_NBUF,)),
        ],
    )
    def k(h_hbm, ei_hbm, z_hbm, out_hbm, src_v, dst_v, rows_v, acc_sh,
          gsem, ssem):
        c = lax.axis_index("c")
        s = lax.axis_index("s")
        wid = s * _NC + c
        pltpu.sync_copy(z_hbm.at[pl.ds(s * _RPT, _RPT)],
                        acc_sh.at[pl.ds(s * _RPT, _RPT)])

        @pl.when(s == 0)
        def _():
            pltpu.sync_copy(z_hbm.at[pl.ds(_RPT * _NS, _RTAIL)],
                            acc_sh.at[pl.ds(_RPT * _NS, _RTAIL)])

        first = wid * _CHUNKS_PER_W + jnp.minimum(wid, _CHUNK_REM)

        pltpu.sync_copy(ei_hbm.at[0, pl.ds(first, _CHUNKS_PER_W)],
                        src_v.at[pl.ds(0, _CHUNKS_PER_W)])
        pltpu.sync_copy(ei_hbm.at[1, pl.ds(first, _CHUNKS_PER_W)],
                        dst_v.at[pl.ds(0, _CHUNKS_PER_W)])

        @pl.when(wid < _CHUNK_REM)
        def _():
            pltpu.sync_copy(ei_hbm.at[0, pl.ds(first + _CHUNKS_PER_W, 1)],
                            src_v.at[pl.ds(_CHUNKS_PER_W, 1)])
            pltpu.sync_copy(ei_hbm.at[1, pl.ds(first + _CHUNKS_PER_W, 1)],
                            dst_v.at[pl.ds(_CHUNKS_PER_W, 1)])

        plsc.subcore_barrier()

        for b in range(_NBUF):
            pltpu.async_copy(h_hbm.at[src_v.at[b]], rows_v.at[b],
                             gsem.at[b])

        n_outer = _CHUNKS_PER_W // _NBUF

        def body(j, carry):
            i0 = j * _NBUF
            for b in range(_NBUF):
                pltpu.make_async_copy(h_hbm.at[src_v.at[i0 + b]],
                                      rows_v.at[b], gsem.at[b]).wait()
                pltpu.async_copy(rows_v.at[b], acc_sh.at[dst_v.at[i0 + b]],
                                 ssem.at[b], add=True)

            @pl.when(j + 1 < n_outer)
            def _():
                for b in range(_NBUF):
                    pltpu.make_async_copy(rows_v.at[b],
                                          acc_sh.at[dst_v.at[0]],
                                          ssem.at[b]).wait()
                    pltpu.async_copy(h_hbm.at[src_v.at[i0 + _NBUF + b]],
                                     rows_v.at[b], gsem.at[b])

            return carry

        lax.fori_loop(0, n_outer, body, 0)
        for b in range(_NBUF):
            pltpu.make_async_copy(rows_v.at[b], acc_sh.at[dst_v.at[0]],
                                  ssem.at[b]).wait()

        @pl.when(wid < _CHUNK_REM)
        def _():
            i = _CHUNKS_PER_W
            pltpu.async_copy(h_hbm.at[src_v.at[i]], rows_v.at[0],
                             gsem.at[0]).wait()
            pltpu.sync_copy(rows_v.at[0], acc_sh.at[dst_v.at[i]], add=True)

        plsc.subcore_barrier()
        pltpu.sync_copy(acc_sh.at[pl.ds(s * _RPT, _RPT)],
                        out_hbm.at[c, pl.ds(s * _RPT, _RPT)])

        @pl.when(s == 0)
        def _():
            pltpu.sync_copy(acc_sh.at[pl.ds(_RPT * _NS, _RTAIL)],
                            out_hbm.at[c, pl.ds(_RPT * _NS, _RTAIL)])

    return k(h, ei3, zeros)


_NP = N_NODES // 2


def _blockdiag(w):
    i, o = w.shape
    z = jnp.zeros((i, o), jnp.float32)
    return jnp.concatenate(
        [jnp.concatenate([w, z], axis=1), jnp.concatenate([z, w], axis=1)],
        axis=0)


def _tile2(b):
    return jnp.concatenate([b, b]).reshape(1, -1)


def _tc_feat(x2, w2, b2):
    def body(x_ref, w_ref, b_ref, o_ref):
        o_ref[...] = jnp.maximum(
            jnp.dot(x_ref[...], w_ref[...], preferred_element_type=jnp.float32)
            + b_ref[...], 0.0)

    return pl.pallas_call(
        body,
        out_shape=jax.ShapeDtypeStruct((_NP, 2 * HID), jnp.float32),
    )(x2, w2, b2)


def _tc_mlp(h2, agg2, w1d, b1d, w2d, b2d, s3d, b3d, epsp):
    def body(h_ref, a_ref, w1_ref, b1_ref, w2_ref, b2_ref, s3_ref, b3_ref,
             e_ref, o_ref):
        z = h_ref[...] * e_ref[0, 0] + a_ref[0] + a_ref[1]
        z = jnp.maximum(
            jnp.dot(z, w1_ref[...], preferred_element_type=jnp.float32)
            + b1_ref[...], 0.0)
        z = jnp.maximum(
            jnp.dot(z, w2_ref[...], preferred_element_type=jnp.float32)
            + b2_ref[...], 0.0)
        o_ref[...] = jnp.maximum(z * s3_ref[...] + b3_ref[...], 0.0)

    return pl.pallas_call(
        body,
        out_shape=jax.ShapeDtypeStruct((_NP, 2 * HID), jnp.float32),
    )(h2, agg2, w1d, b1d, w2d, b2d, s3d, b3d, epsp)


def _tc_head(h2, batch2, a1wd, a1bd, a2wd, a2bd, a3wd, a3bd, tempp,
             c1w, c1b, c2w, c2b):
    def body(h_ref, b_ref, a1w_ref, a1b_ref, a2w_ref, a2b_ref, a3w_ref,
             a3b_ref, t_ref, c1w_ref, c1b_ref, c2w_ref, c2b_ref,
             lo_ref, sc_ref):
        h_ = h_ref[...]
        a = jnp.dot(h_, a1w_ref[...], preferred_element_type=jnp.float32) \
            + a1b_ref[...]
        a = jnp.where(a > 0, a, 0.2 * a)
        a = jnp.dot(a, a2w_ref[...], preferred_element_type=jnp.float32) \
            + a2b_ref[...]
        a = jnp.where(a > 0, a, 0.2 * a)
        lg = jnp.dot(a, a3w_ref[...], preferred_element_type=jnp.float32) \
            + a3b_ref[...]
        lg = jnp.clip(lg, -10.0, 10.0)
        s = jax.nn.sigmoid(lg * t_ref[0, 0])
        sc_ref[...] = s
        se = jnp.broadcast_to(s[:, 0:1], (_NP, HID))
        so = jnp.broadcast_to(s[:, 1:2], (_NP, HID))
        hs = h_ * jnp.concatenate([se, so], axis=1)
        iota = lax.broadcasted_iota(jnp.int32, (_NP, NGRAPHS), 1)
        oh_e = (b_ref[:, 0:1] == iota).astype(jnp.float32)
        oh_o = (b_ref[:, 1:2] == iota).astype(jnp.float32)
        ge = lax.dot_general(oh_e, hs[:, :HID], (((0,), (0,)), ((), ())),
                             preferred_element_type=jnp.float32)
        ge = ge + lax.dot_general(oh_o, hs[:, HID:], (((0,), (0,)), ((), ())),
                                  preferred_element_type=jnp.float32)
        c1 = jnp.maximum(
            jnp.dot(ge, c1w_ref[...], preferred_element_type=jnp.float32)
            + c1b_ref[...], 0.0)
        lo_ref[...] = jnp.dot(c1, c2w_ref[...],
                              preferred_element_type=jnp.float32) + c2b_ref[...]

    return pl.pallas_call(
        body,
        out_shape=(
            jax.ShapeDtypeStruct((NGRAPHS, OUT), jnp.float32),
            jax.ShapeDtypeStruct((_NP, 2), jnp.float32),
        ),
    )(h2, batch2, a1wd, a1bd, a2wd, a2bd, a3wd, a3bd, tempp, c1w, c1b,
      c2w, c2b)


def kernel(x, edge_index, batch, params):
    ei = edge_index.astype(jnp.int32).reshape(2, _NCHUNKS, _CH)
    bn_s = 1.0 / jnp.sqrt(1.0 + BN_EPS)

    x2 = x.reshape(_NP, 2 * IN_DIM)
    h2 = _tc_feat(x2, _blockdiag(params["feat"]["w"]),
                  _tile2(params["feat"]["b"]))

    zeros = jnp.zeros((N_NODES, HID), jnp.float32)
    for c in params["convs"]:
        g1 = c["bn1g"] * bn_s
        w1 = c["lin1"]["w"] * g1[None, :]
        b1 = c["lin1"]["b"] * g1 + c["bn1b"]
        g2 = c["bn2g"] * bn_s
        w2 = c["lin2"]["w"] * g2[None, :]
        b2 = c["lin2"]["b"] * g2 + c["bn2b"]
        s3 = _tile2(c["bng"] * bn_s)
        b3 = _tile2(c["bnb"])
        epsp = (1.0 + c["eps"]).reshape(1, 1)

        agg = _segsum_sc(h2.reshape(N_NODES, HID), ei, zeros)
        h2 = _tc_mlp(h2, agg.reshape(_NC, _NP, 2 * HID),
                     _blockdiag(w1), _tile2(b1),
                     _blockdiag(w2), _tile2(b2), s3, b3, epsp)

    a3w = params["attn3"]["w"]
    logits, scores2 = _tc_head(
        h2,
        batch.reshape(_NP, 2),
        _blockdiag(params["attn1"]["w"]), _tile2(params["attn1"]["b"]),
        _blockdiag(params["attn2"]["w"]), _tile2(params["attn2"]["b"]),
        _blockdiag(a3w), _tile2(params["attn3"]["b"]),
        params["temp"].reshape(1, 1),
        params["cls1"]["w"], params["cls1"]["b"].reshape(1, HID // 2),
        params["cls2"]["w"], params["cls2"]["b"].reshape(1, OUT),
    )
    return logits, scores2.reshape(N_NODES, 1)


# speedup vs baseline: 17.7449x; 1.0157x over previous
import functools

import jax
import jax.numpy as jnp
from jax import lax
from jax.experimental import pallas as pl
from jax.experimental.pallas import tpu as pltpu
from jax.experimental.pallas import tpu_sc as plsc

N_NODES = 10000
N_EDGES = 320000
IN_DIM = 128
HID = 64
OUT = 3
NLAYERS = 3
NGRAPHS = 128
BN_EPS = 1e-5

_NC = 2
_NS = 16
_NW = _NC * _NS
_CH = 128
_NCHUNKS = N_EDGES // _CH
_CHUNKS_PER_W = _NCHUNKS // _NW
_CHUNK_REM = _NCHUNKS - _CHUNKS_PER_W * _NW
_RPT = 624
_RTAIL = N_NODES - _RPT * _NS


_NBUF = 8


def _segsum_sc(h, ei3, zeros):
    mesh = plsc.VectorSubcoreMesh(core_axis_name="c", subcore_axis_name="s")

    @functools.partial(
        pl.kernel,
        mesh=mesh,
        compiler_params=pltpu.CompilerParams(use_tc_tiling_on_sc=False),
        out_type=jax.ShapeDtypeStruct((_NC, N_NODES, HID), jnp.float32),
        scratch_types=[
            pltpu.VMEM((_CHUNKS_PER_W + 1, _CH), jnp.int32),
            pltpu.VMEM((_CHUNKS_PER_W + 1, _CH), jnp.int32),
            pltpu.VMEM((_NBUF, _CH, HID), jnp.float32),
            pltpu.VMEM_SHARED((N_NODES, HID), jnp.float32),
            pltpu.SemaphoreType.DMA((_NBUF,)),
            pltpu.SemaphoreType.DMA((_NBUF,)),
        ],
    )
    def k(h_hbm, ei_hbm, z_hbm, out_hbm, src_v, dst_v, rows_v, acc_sh,
          gsem, ssem):
        c = lax.axis_index("c")
        s = lax.axis_index("s")
        wid = s * _NC + c
        pltpu.sync_copy(z_hbm.at[pl.ds(s * _RPT, _RPT)],
                        acc_sh.at[pl.ds(s * _RPT, _RPT)])

        @pl.when(s == 0)
        def _():
            pltpu.sync_copy(z_hbm.at[pl.ds(_RPT * _NS, _RTAIL)],
                            acc_sh.at[pl.ds(_RPT * _NS, _RTAIL)])

        first = wid * _CHUNKS_PER_W + jnp.minimum(wid, _CHUNK_REM)

        pltpu.sync_copy(ei_hbm.at[0, pl.ds(first, _CHUNKS_PER_W)],
                        src_v.at[pl.ds(0, _CHUNKS_PER_W)])
        pltpu.sync_copy(ei_hbm.at[1, pl.ds(first, _CHUNKS_PER_W)],
                        dst_v.at[pl.ds(0, _CHUNKS_PER_W)])

        @pl.when(wid < _CHUNK_REM)
        def _():
            pltpu.sync_copy(ei_hbm.at[0, pl.ds(first + _CHUNKS_PER_W, 1)],
                            src_v.at[pl.ds(_CHUNKS_PER_W, 1)])
            pltpu.sync_copy(ei_hbm.at[1, pl.ds(first + _CHUNKS_PER_W, 1)],
                            dst_v.at[pl.ds(_CHUNKS_PER_W, 1)])

        plsc.subcore_barrier()

        for b in range(_NBUF):
            pltpu.async_copy(h_hbm.at[src_v.at[b]], rows_v.at[b],
                             gsem.at[b])

        n_outer = -(-_CHUNKS_PER_W // _NBUF)

        def body(j, carry):
            i0 = j * _NBUF
            for b in range(_NBUF):
                @pl.when(i0 + b < _CHUNKS_PER_W)
                def _(b=b):
                    pltpu.make_async_copy(h_hbm.at[src_v.at[i0 + b]],
                                          rows_v.at[b], gsem.at[b]).wait()
                    pltpu.async_copy(rows_v.at[b],
                                     acc_sh.at[dst_v.at[i0 + b]],
                                     ssem.at[b], add=True)

            for b in range(_NBUF):
                @pl.when(i0 + _NBUF + b < _CHUNKS_PER_W)
                def _(b=b):
                    pltpu.make_async_copy(rows_v.at[b],
                                          acc_sh.at[dst_v.at[0]],
                                          ssem.at[b]).wait()
                    pltpu.async_copy(h_hbm.at[src_v.at[i0 + _NBUF + b]],
                                     rows_v.at[b], gsem.at[b])

            return carry

        lax.fori_loop(0, n_outer, body, 0)
        for b in range(_NBUF):
            pltpu.make_async_copy(rows_v.at[b], acc_sh.at[dst_v.at[0]],
                                  ssem.at[b]).wait()

        @pl.when(wid < _CHUNK_REM)
        def _():
            i = _CHUNKS_PER_W
            pltpu.async_copy(h_hbm.at[src_v.at[i]], rows_v.at[0],
                             gsem.at[0]).wait()
            pltpu.sync_copy(rows_v.at[0], acc_sh.at[dst_v.at[i]], add=True)

        plsc.subcore_barrier()
        pltpu.sync_copy(acc_sh.at[pl.ds(s * _RPT, _RPT)],
                        out_hbm.at[c, pl.ds(s * _RPT, _RPT)])

        @pl.when(s == 0)
        def _():
            pltpu.sync_copy(acc_sh.at[pl.ds(_RPT * _NS, _RTAIL)],
                            out_hbm.at[c, pl.ds(_RPT * _NS, _RTAIL)])

    return k(h, ei3, zeros)


_NP = N_NODES // 2


def _blockdiag(w):
    i, o = w.shape
    z = jnp.zeros((i, o), jnp.float32)
    return jnp.concatenate(
        [jnp.concatenate([w, z], axis=1), jnp.concatenate([z, w], axis=1)],
        axis=0)


def _tile2(b):
    return jnp.concatenate([b, b]).reshape(1, -1)


def _tc_feat(x2, w2, b2):
    def body(x_ref, w_ref, b_ref, o_ref):
        o_ref[...] = jnp.maximum(
            jnp.dot(x_ref[...], w_ref[...], preferred_element_type=jnp.float32)
            + b_ref[...], 0.0)

    return pl.pallas_call(
        body,
        out_shape=jax.ShapeDtypeStruct((_NP, 2 * HID), jnp.float32),
    )(x2, w2, b2)


def _tc_mlp(h2, agg2, w1d, b1d, w2d, b2d, s3d, b3d, epsp):
    def body(h_ref, a_ref, w1_ref, b1_ref, w2_ref, b2_ref, s3_ref, b3_ref,
             e_ref, o_ref):
        z = h_ref[...] * e_ref[0, 0] + a_ref[0] + a_ref[1]
        z = jnp.maximum(
            jnp.dot(z, w1_ref[...], preferred_element_type=jnp.float32)
            + b1_ref[...], 0.0)
        z = jnp.maximum(
            jnp.dot(z, w2_ref[...], preferred_element_type=jnp.float32)
            + b2_ref[...], 0.0)
        o_ref[...] = jnp.maximum(z * s3_ref[...] + b3_ref[...], 0.0)

    return pl.pallas_call(
        body,
        out_shape=jax.ShapeDtypeStruct((_NP, 2 * HID), jnp.float32),
    )(h2, agg2, w1d, b1d, w2d, b2d, s3d, b3d, epsp)


def _tc_head(h2, batch2, a1wd, a1bd, a2wd, a2bd, a3wd, a3bd, tempp,
             c1w, c1b, c2w, c2b):
    def body(h_ref, b_ref, a1w_ref, a1b_ref, a2w_ref, a2b_ref, a3w_ref,
             a3b_ref, t_ref, c1w_ref, c1b_ref, c2w_ref, c2b_ref,
             lo_ref, sc_ref):
        h_ = h_ref[...]
        a = jnp.dot(h_, a1w_ref[...], preferred_element_type=jnp.float32) \
            + a1b_ref[...]
        a = jnp.where(a > 0, a, 0.2 * a)
        a = jnp.dot(a, a2w_ref[...], preferred_element_type=jnp.float32) \
            + a2b_ref[...]
        a = jnp.where(a > 0, a, 0.2 * a)
        lg = jnp.dot(a, a3w_ref[...], preferred_element_type=jnp.float32) \
            + a3b_ref[...]
        lg = jnp.clip(lg, -10.0, 10.0)
        s = jax.nn.sigmoid(lg * t_ref[0, 0])
        sc_ref[...] = s
        se = jnp.broadcast_to(s[:, 0:1], (_NP, HID))
        so = jnp.broadcast_to(s[:, 1:2], (_NP, HID))
        hs = h_ * jnp.concatenate([se, so], axis=1)
        iota = lax.broadcasted_iota(jnp.int32, (_NP, NGRAPHS), 1)
        oh_e = (b_ref[:, 0:1] == iota).astype(jnp.float32)
        oh_o = (b_ref[:, 1:2] == iota).astype(jnp.float32)
        ge = lax.dot_general(oh_e, hs[:, :HID], (((0,), (0,)), ((), ())),
                             preferred_element_type=jnp.float32)
        ge = ge + lax.dot_general(oh_o, hs[:, HID:], (((0,), (0,)), ((), ())),
                                  preferred_element_type=jnp.float32)
        c1 = jnp.maximum(
            jnp.dot(ge, c1w_ref[...], preferred_element_type=jnp.float32)
            + c1b_ref[...], 0.0)
        lo_ref[...] = jnp.dot(c1, c2w_ref[...],
                              preferred_element_type=jnp.float32) + c2b_ref[...]

    return pl.pallas_call(
        body,
        out_shape=(
            jax.ShapeDtypeStruct((NGRAPHS, OUT), jnp.float32),
            jax.ShapeDtypeStruct((_NP, 2), jnp.float32),
        ),
    )(h2, batch2, a1wd, a1bd, a2wd, a2bd, a3wd, a3bd, tempp, c1w, c1b,
      c2w, c2b)


def kernel(x, edge_index, batch, params):
    ei = edge_index.astype(jnp.int32).reshape(2, _NCHUNKS, _CH)
    bn_s = 1.0 / jnp.sqrt(1.0 + BN_EPS)

    x2 = x.reshape(_NP, 2 * IN_DIM)
    h2 = _tc_feat(x2, _blockdiag(params["feat"]["w"]),
                  _tile2(params["feat"]["b"]))

    zeros = jnp.zeros((N_NODES, HID), jnp.float32)
    for c in params["convs"]:
        g1 = c["bn1g"] * bn_s
        w1 = c["lin1"]["w"] * g1[None, :]
        b1 = c["lin1"]["b"] * g1 + c["bn1b"]
        g2 = c["bn2g"] * bn_s
        w2 = c["lin2"]["w"] * g2[None, :]
        b2 = c["lin2"]["b"] * g2 + c["bn2b"]
        s3 = _tile2(c["bng"] * bn_s)
        b3 = _tile2(c["bnb"])
        epsp = (1.0 + c["eps"]).reshape(1, 1)

        agg = _segsum_sc(h2.reshape(N_NODES, HID), ei, zeros)
        h2 = _tc_mlp(h2, agg.reshape(_NC, _NP, 2 * HID),
                     _blockdiag(w1), _tile2(b1),
                     _blockdiag(w2), _tile2(b2), s3, b3, epsp)

    a3w = params["attn3"]["w"]
    logits, scores2 = _tc_head(
        h2,
        batch.reshape(_NP, 2),
        _blockdiag(params["attn1"]["w"]), _tile2(params["attn1"]["b"]),
        _blockdiag(params["attn2"]["w"]), _tile2(params["attn2"]["b"]),
        _blockdiag(a3w), _tile2(params["attn3"]["b"]),
        params["temp"].reshape(1, 1),
        params["cls1"]["w"], params["cls1"]["b"].reshape(1, HID // 2),
        params["cls2"]["w"], params["cls2"]["b"].reshape(1, OUT),
    )
    return logits, scores2.reshape(N_NODES, 1)
